# baseline (speedup 1.0000x reference)
import functools

import jax
import jax.numpy as jnp
from jax import lax
from jax.experimental import pallas as pl
from jax.experimental.pallas import tpu as pltpu
from jax.experimental.pallas import tpu_sc as plsc

N_E = 8192
E_DIM = 256
B = 16
T = 1024
JT = 256
NJ = N_E // JT


def _argmin_body(z_ref, emb_ref, idx_ref, loss_ref,
                 z2_scr, runmax_scr, runidx_scr, loss_scr):
    b = pl.program_id(0)
    j = pl.program_id(1)

    zb = z_ref[0]

    @pl.when(j == 0)
    def _():
        z2_scr[...] = jnp.sum(zb * zb, axis=0, keepdims=True)

    m = lax.dot_general(emb_ref[...].astype(jnp.bfloat16),
                        zb.astype(jnp.bfloat16),
                        dimension_numbers=(((1,), (0,)), ((), ())),
                        preferred_element_type=jnp.float32)

    tmax = jnp.max(m, axis=0, keepdims=True)
    iot = lax.broadcasted_iota(jnp.int32, (JT, T), 0) + j * JT
    tidx = jnp.min(jnp.where(m == tmax, iot, jnp.int32(2**30)),
                   axis=0, keepdims=True)

    @pl.when(j == 0)
    def _():
        runmax_scr[...] = tmax
        runidx_scr[...] = tidx

    @pl.when(j > 0)
    def _():
        upd = tmax > runmax_scr[...]
        runidx_scr[...] = jnp.where(upd, tidx, runidx_scr[...])
        runmax_scr[...] = jnp.where(upd, tmax, runmax_scr[...])

    @pl.when(j == NJ - 1)
    def _():
        idx_ref[0] = runidx_scr[...]

        @pl.when(b == 0)
        def _():
            loss_scr[0, 0] = 0.0

        loss_scr[0, 0] += jnp.sum(z2_scr[...] - 2.0 * runmax_scr[...])

        @pl.when(b == B - 1)
        def _():
            loss_ref[...] = jnp.full(
                (1, 1), 1.25 * loss_scr[0, 0] / (B * T * E_DIM), jnp.float32)


def _argmin_call(z, emb):
    return pl.pallas_call(
        _argmin_body,
        grid=(B, NJ),
        in_specs=[
            pl.BlockSpec((1, E_DIM, T), lambda b, j: (b, 0, 0)),
            pl.BlockSpec((JT, E_DIM), lambda b, j: (j, 0)),
        ],
        out_specs=[
            pl.BlockSpec((1, 1, T), lambda b, j: (b, 0, 0)),
            pl.BlockSpec((1, 1), lambda b, j: (0, 0)),
        ],
        out_shape=[
            jax.ShapeDtypeStruct((B, 1, T), jnp.int32),
            jax.ShapeDtypeStruct((1, 1), jnp.float32),
        ],
        scratch_shapes=[
            pltpu.VMEM((1, T), jnp.float32),
            pltpu.VMEM((1, T), jnp.float32),
            pltpu.VMEM((1, T), jnp.int32),
            pltpu.SMEM((1, 1), jnp.float32),
        ],
    )(z, emb)


def _make_sc_gather():
    try:
        info = plsc.get_sparse_core_info()
        NC, NS = info.num_cores, info.num_subcores
    except Exception:
        NC, NS = 2, 16
    NW = NC * NS
    n_rows = B * T
    b_per_w = n_rows // NW
    CH = 128
    n_ch = b_per_w // CH

    mesh = plsc.VectorSubcoreMesh(core_axis_name="c", subcore_axis_name="s",
                                  num_cores=NC)

    @functools.partial(
        pl.kernel, mesh=mesh,
        out_type=jax.ShapeDtypeStruct((n_rows, E_DIM), jnp.float32),
        scratch_types=[
            pltpu.VMEM((CH,), jnp.int32),
            pltpu.VMEM((CH, E_DIM), jnp.float32),
            pltpu.SemaphoreType.DMA,
        ],
    )
    def gather_k(emb_hbm, idx_hbm, out_hbm, idx_v, rows_v, sem):
        wid = lax.axis_index("s") * NC + lax.axis_index("c")
        base = wid * b_per_w
        for c in range(n_ch):
            off = base + c * CH
            pltpu.sync_copy(idx_hbm.at[pl.ds(off, CH)], idx_v)
            pltpu.async_copy(emb_hbm.at[idx_v], rows_v, sem).wait()
            pltpu.sync_copy(rows_v, out_hbm.at[pl.ds(off, CH)])

    return gather_k


@functools.cache
def _sc_gather_cached():
    return _make_sc_gather()


def _transpose_body(rows_ref, out_ref):
    out_ref[0] = rows_ref[0].T


def _transpose_call(rows):
    return pl.pallas_call(
        _transpose_body,
        grid=(B,),
        in_specs=[pl.BlockSpec((1, T, E_DIM), lambda b: (b, 0, 0))],
        out_specs=pl.BlockSpec((1, E_DIM, T), lambda b: (b, 0, 0)),
        out_shape=jax.ShapeDtypeStruct((B, E_DIM, T), jnp.float32),
    )(rows)


def kernel(z, emb):
    idx3, loss = _argmin_call(z, emb)
    idx = idx3.reshape(B, T)
    rows = _sc_gather_cached()(emb, idx.reshape(B * T))
    zq = _transpose_call(rows.reshape(B, T, E_DIM))
    return zq, loss.reshape(()), idx


# speedup vs baseline: 1.7195x; 1.7195x over previous
import functools

import jax
import jax.numpy as jnp
from jax import lax
from jax.experimental import pallas as pl
from jax.experimental.pallas import tpu as pltpu
from jax.experimental.pallas import tpu_sc as plsc

N_E = 8192
E_DIM = 256
B = 16
T = 1024
JT = 1024
NJ = N_E // JT


def _argmin_body(z_ref, emb_ref, idx_ref, loss_ref,
                 z2_scr, runmax_scr, runidx_scr, loss_scr):
    b = pl.program_id(0)
    j = pl.program_id(1)

    zb = z_ref[0]

    @pl.when(j == 0)
    def _():
        z2_scr[...] = jnp.sum(zb * zb, axis=0, keepdims=True)

    m = lax.dot_general(emb_ref[...].astype(jnp.bfloat16),
                        zb.astype(jnp.bfloat16),
                        dimension_numbers=(((1,), (0,)), ((), ())),
                        preferred_element_type=jnp.float32)

    tmax = jnp.max(m, axis=0, keepdims=True)
    iot = lax.broadcasted_iota(jnp.int32, (JT, T), 0) + j * JT
    tidx = jnp.min(jnp.where(m == tmax, iot, jnp.int32(2**30)),
                   axis=0, keepdims=True)

    @pl.when(j == 0)
    def _():
        runmax_scr[...] = tmax
        runidx_scr[...] = tidx

    @pl.when(j > 0)
    def _():
        upd = tmax > runmax_scr[...]
        runidx_scr[...] = jnp.where(upd, tidx, runidx_scr[...])
        runmax_scr[...] = jnp.where(upd, tmax, runmax_scr[...])

    @pl.when(j == NJ - 1)
    def _():
        idx_ref[0] = runidx_scr[...]

        @pl.when(b == 0)
        def _():
            loss_scr[0, 0] = 0.0

        loss_scr[0, 0] += jnp.sum(z2_scr[...] - 2.0 * runmax_scr[...])

        @pl.when(b == B - 1)
        def _():
            loss_ref[...] = jnp.full(
                (1, 1), 1.25 * loss_scr[0, 0] / (B * T * E_DIM), jnp.float32)


def _argmin_call(z, emb):
    return pl.pallas_call(
        _argmin_body,
        grid=(B, NJ),
        in_specs=[
            pl.BlockSpec((1, E_DIM, T), lambda b, j: (b, 0, 0)),
            pl.BlockSpec((JT, E_DIM), lambda b, j: (j, 0)),
        ],
        out_specs=[
            pl.BlockSpec((1, 1, T), lambda b, j: (b, 0, 0)),
            pl.BlockSpec((1, 1), lambda b, j: (0, 0)),
        ],
        out_shape=[
            jax.ShapeDtypeStruct((B, 1, T), jnp.int32),
            jax.ShapeDtypeStruct((1, 1), jnp.float32),
        ],
        scratch_shapes=[
            pltpu.VMEM((1, T), jnp.float32),
            pltpu.VMEM((1, T), jnp.float32),
            pltpu.VMEM((1, T), jnp.int32),
            pltpu.SMEM((1, 1), jnp.float32),
        ],
    )(z, emb)


def _make_sc_gather():
    try:
        info = plsc.get_sparse_core_info()
        NC, NS = info.num_cores, info.num_subcores
    except Exception:
        NC, NS = 2, 16
    NW = NC * NS
    n_rows = B * T
    b_per_w = n_rows // NW
    CH = 128
    n_ch = b_per_w // CH

    mesh = plsc.VectorSubcoreMesh(core_axis_name="c", subcore_axis_name="s",
                                  num_cores=NC)

    @functools.partial(
        pl.kernel, mesh=mesh,
        out_type=jax.ShapeDtypeStruct((n_rows, E_DIM), jnp.float32),
        scratch_types=[
            pltpu.VMEM((CH,), jnp.int32),
            pltpu.VMEM((CH, E_DIM), jnp.float32),
            pltpu.SemaphoreType.DMA,
        ],
    )
    def gather_k(emb_hbm, idx_hbm, out_hbm, idx_v, rows_v, sem):
        wid = lax.axis_index("s") * NC + lax.axis_index("c")
        base = wid * b_per_w
        for c in range(n_ch):
            off = base + c * CH
            pltpu.sync_copy(idx_hbm.at[pl.ds(off, CH)], idx_v)
            pltpu.async_copy(emb_hbm.at[idx_v], rows_v, sem).wait()
            pltpu.sync_copy(rows_v, out_hbm.at[pl.ds(off, CH)])

    return gather_k


@functools.cache
def _sc_gather_cached():
    return _make_sc_gather()


def _transpose_body(rows_ref, out_ref):
    out_ref[0] = rows_ref[0].T


def _transpose_call(rows):
    return pl.pallas_call(
        _transpose_body,
        grid=(B,),
        in_specs=[pl.BlockSpec((1, T, E_DIM), lambda b: (b, 0, 0))],
        out_specs=pl.BlockSpec((1, E_DIM, T), lambda b: (b, 0, 0)),
        out_shape=jax.ShapeDtypeStruct((B, E_DIM, T), jnp.float32),
    )(rows)


def kernel(z, emb):
    idx3, loss = _argmin_call(z, emb)
    idx = idx3.reshape(B, T)
    rows = _sc_gather_cached()(emb, idx.reshape(B * T))
    zq = _transpose_call(rows.reshape(B, T, E_DIM))
    return zq, loss.reshape(()), idx


# speedup vs baseline: 2.0340x; 1.1829x over previous
import functools

import jax
import jax.numpy as jnp
from jax import lax
from jax.experimental import pallas as pl
from jax.experimental.pallas import tpu as pltpu
from jax.experimental.pallas import tpu_sc as plsc

N_E = 8192
E_DIM = 256
B = 16
T = 1024
JT = 2048
NJ = N_E // JT


def _argmin_body(z_ref, emb_ref, idx_ref, loss_ref,
                 z2_scr, runmax_scr, runidx_scr, loss_scr):
    b = pl.program_id(0)
    j = pl.program_id(1)

    zb = z_ref[0]

    @pl.when(j == 0)
    def _():
        z2_scr[...] = jnp.sum(zb * zb, axis=0, keepdims=True)

    m = lax.dot_general(emb_ref[...].astype(jnp.bfloat16),
                        zb.astype(jnp.bfloat16),
                        dimension_numbers=(((1,), (0,)), ((), ())),
                        preferred_element_type=jnp.float32)

    tmax = jnp.max(m, axis=0, keepdims=True)
    iot = lax.broadcasted_iota(jnp.int32, (JT, T), 0) + j * JT
    tidx = jnp.min(jnp.where(m == tmax, iot, jnp.int32(2**30)),
                   axis=0, keepdims=True)

    @pl.when(j == 0)
    def _():
        runmax_scr[...] = tmax
        runidx_scr[...] = tidx

    @pl.when(j > 0)
    def _():
        upd = tmax > runmax_scr[...]
        runidx_scr[...] = jnp.where(upd, tidx, runidx_scr[...])
        runmax_scr[...] = jnp.where(upd, tmax, runmax_scr[...])

    @pl.when(j == NJ - 1)
    def _():
        idx_ref[0] = runidx_scr[...]

        @pl.when(b == 0)
        def _():
            loss_scr[0, 0] = 0.0

        loss_scr[0, 0] += jnp.sum(z2_scr[...] - 2.0 * runmax_scr[...])

        @pl.when(b == B - 1)
        def _():
            loss_ref[...] = jnp.full(
                (1, 1), 1.25 * loss_scr[0, 0] / (B * T * E_DIM), jnp.float32)


def _argmin_call(z, emb):
    return pl.pallas_call(
        _argmin_body,
        grid=(B, NJ),
        in_specs=[
            pl.BlockSpec((1, E_DIM, T), lambda b, j: (b, 0, 0)),
            pl.BlockSpec((JT, E_DIM), lambda b, j: (j, 0)),
        ],
        out_specs=[
            pl.BlockSpec((1, 1, T), lambda b, j: (b, 0, 0)),
            pl.BlockSpec((1, 1), lambda b, j: (0, 0)),
        ],
        out_shape=[
            jax.ShapeDtypeStruct((B, 1, T), jnp.int32),
            jax.ShapeDtypeStruct((1, 1), jnp.float32),
        ],
        scratch_shapes=[
            pltpu.VMEM((1, T), jnp.float32),
            pltpu.VMEM((1, T), jnp.float32),
            pltpu.VMEM((1, T), jnp.int32),
            pltpu.SMEM((1, 1), jnp.float32),
        ],
    )(z, emb)


def _make_sc_gather():
    try:
        info = plsc.get_sparse_core_info()
        NC, NS = info.num_cores, info.num_subcores
    except Exception:
        NC, NS = 2, 16
    NW = NC * NS
    n_rows = B * T
    b_per_w = n_rows // NW
    CH = 128
    n_ch = b_per_w // CH

    mesh = plsc.VectorSubcoreMesh(core_axis_name="c", subcore_axis_name="s",
                                  num_cores=NC)

    @functools.partial(
        pl.kernel, mesh=mesh,
        out_type=jax.ShapeDtypeStruct((n_rows, E_DIM), jnp.float32),
        scratch_types=[
            pltpu.VMEM((CH,), jnp.int32),
            pltpu.VMEM((CH, E_DIM), jnp.float32),
            pltpu.SemaphoreType.DMA,
        ],
    )
    def gather_k(emb_hbm, idx_hbm, out_hbm, idx_v, rows_v, sem):
        wid = lax.axis_index("s") * NC + lax.axis_index("c")
        base = wid * b_per_w
        for c in range(n_ch):
            off = base + c * CH
            pltpu.sync_copy(idx_hbm.at[pl.ds(off, CH)], idx_v)
            pltpu.async_copy(emb_hbm.at[idx_v], rows_v, sem).wait()
            pltpu.sync_copy(rows_v, out_hbm.at[pl.ds(off, CH)])

    return gather_k


@functools.cache
def _sc_gather_cached():
    return _make_sc_gather()


def _transpose_body(rows_ref, out_ref):
    out_ref[0] = rows_ref[0].T


def _transpose_call(rows):
    return pl.pallas_call(
        _transpose_body,
        grid=(B,),
        in_specs=[pl.BlockSpec((1, T, E_DIM), lambda b: (b, 0, 0))],
        out_specs=pl.BlockSpec((1, E_DIM, T), lambda b: (b, 0, 0)),
        out_shape=jax.ShapeDtypeStruct((B, E_DIM, T), jnp.float32),
    )(rows)


def kernel(z, emb):
    idx3, loss = _argmin_call(z, emb)
    idx = idx3.reshape(B, T)
    rows = _sc_gather_cached()(emb, idx.reshape(B * T))
    zq = _transpose_call(rows.reshape(B, T, E_DIM))
    return zq, loss.reshape(()), idx


# speedup vs baseline: 2.1465x; 1.0553x over previous
import functools

import jax
import jax.numpy as jnp
from jax import lax
from jax.experimental import pallas as pl
from jax.experimental.pallas import tpu as pltpu
from jax.experimental.pallas import tpu_sc as plsc

N_E = 8192
E_DIM = 256
B = 16
T = 1024
JT = 4096
NJ = N_E // JT


def _argmin_body(z_ref, emb_ref, idx_ref, loss_ref,
                 z2_scr, runmax_scr, runidx_scr, loss_scr):
    b = pl.program_id(0)
    j = pl.program_id(1)

    zb = z_ref[0]

    @pl.when(j == 0)
    def _():
        z2_scr[...] = jnp.sum(zb * zb, axis=0, keepdims=True)

    m = lax.dot_general(emb_ref[...].astype(jnp.bfloat16),
                        zb.astype(jnp.bfloat16),
                        dimension_numbers=(((1,), (0,)), ((), ())),
                        preferred_element_type=jnp.float32)

    tmax = jnp.max(m, axis=0, keepdims=True)
    iot = lax.broadcasted_iota(jnp.int32, (JT, T), 0) + j * JT
    tidx = jnp.min(jnp.where(m == tmax, iot, jnp.int32(2**30)),
                   axis=0, keepdims=True)

    @pl.when(j == 0)
    def _():
        runmax_scr[...] = tmax
        runidx_scr[...] = tidx

    @pl.when(j > 0)
    def _():
        upd = tmax > runmax_scr[...]
        runidx_scr[...] = jnp.where(upd, tidx, runidx_scr[...])
        runmax_scr[...] = jnp.where(upd, tmax, runmax_scr[...])

    @pl.when(j == NJ - 1)
    def _():
        idx_ref[0] = runidx_scr[...]

        @pl.when(b == 0)
        def _():
            loss_scr[0, 0] = 0.0

        loss_scr[0, 0] += jnp.sum(z2_scr[...] - 2.0 * runmax_scr[...])

        @pl.when(b == B - 1)
        def _():
            loss_ref[...] = jnp.full(
                (1, 1), 1.25 * loss_scr[0, 0] / (B * T * E_DIM), jnp.float32)


def _argmin_call(z, emb):
    return pl.pallas_call(
        _argmin_body,
        grid=(B, NJ),
        in_specs=[
            pl.BlockSpec((1, E_DIM, T), lambda b, j: (b, 0, 0)),
            pl.BlockSpec((JT, E_DIM), lambda b, j: (j, 0)),
        ],
        out_specs=[
            pl.BlockSpec((1, 1, T), lambda b, j: (b, 0, 0)),
            pl.BlockSpec((1, 1), lambda b, j: (0, 0)),
        ],
        out_shape=[
            jax.ShapeDtypeStruct((B, 1, T), jnp.int32),
            jax.ShapeDtypeStruct((1, 1), jnp.float32),
        ],
        scratch_shapes=[
            pltpu.VMEM((1, T), jnp.float32),
            pltpu.VMEM((1, T), jnp.float32),
            pltpu.VMEM((1, T), jnp.int32),
            pltpu.SMEM((1, 1), jnp.float32),
        ],
    )(z, emb)


def _make_sc_gather():
    try:
        info = plsc.get_sparse_core_info()
        NC, NS = info.num_cores, info.num_subcores
    except Exception:
        NC, NS = 2, 16
    NW = NC * NS
    n_rows = B * T
    b_per_w = n_rows // NW
    CH = 128
    n_ch = b_per_w // CH

    mesh = plsc.VectorSubcoreMesh(core_axis_name="c", subcore_axis_name="s",
                                  num_cores=NC)

    @functools.partial(
        pl.kernel, mesh=mesh,
        out_type=jax.ShapeDtypeStruct((n_rows, E_DIM), jnp.float32),
        scratch_types=[
            pltpu.VMEM((CH,), jnp.int32),
            pltpu.VMEM((CH, E_DIM), jnp.float32),
            pltpu.SemaphoreType.DMA,
        ],
    )
    def gather_k(emb_hbm, idx_hbm, out_hbm, idx_v, rows_v, sem):
        wid = lax.axis_index("s") * NC + lax.axis_index("c")
        base = wid * b_per_w
        for c in range(n_ch):
            off = base + c * CH
            pltpu.sync_copy(idx_hbm.at[pl.ds(off, CH)], idx_v)
            pltpu.async_copy(emb_hbm.at[idx_v], rows_v, sem).wait()
            pltpu.sync_copy(rows_v, out_hbm.at[pl.ds(off, CH)])

    return gather_k


@functools.cache
def _sc_gather_cached():
    return _make_sc_gather()


def _transpose_body(rows_ref, out_ref):
    out_ref[0] = rows_ref[0].T


def _transpose_call(rows):
    return pl.pallas_call(
        _transpose_body,
        grid=(B,),
        in_specs=[pl.BlockSpec((1, T, E_DIM), lambda b: (b, 0, 0))],
        out_specs=pl.BlockSpec((1, E_DIM, T), lambda b: (b, 0, 0)),
        out_shape=jax.ShapeDtypeStruct((B, E_DIM, T), jnp.float32),
    )(rows)


def kernel(z, emb):
    idx3, loss = _argmin_call(z, emb)
    idx = idx3.reshape(B, T)
    rows = _sc_gather_cached()(emb, idx.reshape(B * T))
    zq = _transpose_call(rows.reshape(B, T, E_DIM))
    return zq, loss.reshape(()), idx
